# baseline (device time: 11774 ns/iter reference)
import jax
import jax.numpy as jnp
from jax import lax
from jax.experimental import pallas as pl
from jax.experimental.pallas import tpu as pltpu

M = 512
N = 512
H = M // 2
CHUNKS = 4
R = H // CHUNKS


def kernel(x):
    xb = x.reshape(M, N).astype(jnp.bfloat16)

    def body(x_ref, out_ref, rbuf_a1, rbuf_b1, rbuf_a2, rbuf_b2, send_sems, recv_sems):
        my_x = lax.axis_index("x")
        my_y = lax.axis_index("y")
        x_nbr = (1 - my_x, my_y)
        y_nbr = (my_x, 1 - my_y)

        barrier_sem = pltpu.get_barrier_semaphore()
        for nbr in (x_nbr, y_nbr):
            pl.semaphore_signal(
                barrier_sem, inc=1,
                device_id=nbr, device_id_type=pl.DeviceIdType.MESH,
            )
        pl.semaphore_wait(barrier_sem, 2)

        def copy(src, dst, sem_row, k, nbr):
            return pltpu.make_async_remote_copy(
                src_ref=src, dst_ref=dst.at[k],
                send_sem=send_sems.at[sem_row, k],
                recv_sem=recv_sems.at[sem_row, k],
                device_id=nbr, device_id_type=pl.DeviceIdType.MESH,
            )

        a1, b1 = [], []
        for k in range(CHUNKS):
            rdma = copy(x_ref.at[pl.ds(k * R, R)], rbuf_a1, 0, k, x_nbr)
            rdma.start()
            a1.append(rdma)
            rdma = copy(x_ref.at[pl.ds(H + k * R, R)], rbuf_b1, 1, k, y_nbr)
            rdma.start()
            b1.append(rdma)

        a2, b2 = [], []
        for k in range(CHUNKS):
            a1[k].wait()
            out_ref[k * R:(k + 1) * R, :] = (
                x_ref[k * R:(k + 1) * R, :] + rbuf_a1[k]
            )
            rdma = copy(out_ref.at[pl.ds(k * R, R)], rbuf_a2, 2, k, y_nbr)
            rdma.start()
            a2.append(rdma)
            b1[k].wait()
            out_ref[H + k * R:H + (k + 1) * R, :] = (
                x_ref[H + k * R:H + (k + 1) * R, :] + rbuf_b1[k]
            )
            rdma = copy(out_ref.at[pl.ds(H + k * R, R)], rbuf_b2, 3, k, x_nbr)
            rdma.start()
            b2.append(rdma)

        for k in range(CHUNKS):
            a2[k].wait()
            out_ref[k * R:(k + 1) * R, :] = (
                out_ref[k * R:(k + 1) * R, :] + rbuf_a2[k]
            )
            b2[k].wait()
            out_ref[H + k * R:H + (k + 1) * R, :] = (
                out_ref[H + k * R:H + (k + 1) * R, :] + rbuf_b2[k]
            )

    return pl.pallas_call(
        body,
        out_shape=jax.ShapeDtypeStruct((M, N), jnp.bfloat16),
        in_specs=[pl.BlockSpec(memory_space=pltpu.VMEM)],
        out_specs=pl.BlockSpec(memory_space=pltpu.VMEM),
        scratch_shapes=[
            pltpu.VMEM((CHUNKS, R, N), jnp.bfloat16),
            pltpu.VMEM((CHUNKS, R, N), jnp.bfloat16),
            pltpu.VMEM((CHUNKS, R, N), jnp.bfloat16),
            pltpu.VMEM((CHUNKS, R, N), jnp.bfloat16),
            pltpu.SemaphoreType.DMA((4, CHUNKS)),
            pltpu.SemaphoreType.DMA((4, CHUNKS)),
        ],
        compiler_params=pltpu.CompilerParams(collective_id=0),
    )(xb)
